# baseline (speedup 1.0000x reference)
import functools

import jax
import jax.numpy as jnp
from jax import lax
from jax.experimental import pallas as pl
from jax.experimental.pallas import tpu as pltpu
from jax.experimental.pallas import tpu_sc as plsc

_B = 16384
_EMB = 16
_NC, _NS = 2, 16
_NW = _NC * _NS
_RPW = _B // _NW
_CHUNK = 128
_NCHUNK = _RPW // _CHUNK


def _sc_gather(item_idx, user_idx, cat_idx, t_item, t_avg, t_user, t_cat):
    mesh = plsc.VectorSubcoreMesh(core_axis_name="c", subcore_axis_name="s")

    @functools.partial(
        pl.kernel,
        mesh=mesh,
        compiler_params=pltpu.CompilerParams(use_tc_tiling_on_sc=False),
        out_type=[
            jax.ShapeDtypeStruct((_B, _EMB), jnp.float32),
            jax.ShapeDtypeStruct((_B, _EMB), jnp.float32),
            jax.ShapeDtypeStruct((_B, _EMB), jnp.float32),
            jax.ShapeDtypeStruct((_B, _EMB), jnp.float32),
            jax.ShapeDtypeStruct((_NW, _EMB), jnp.float32),
        ],
        scratch_types=[
            pltpu.VMEM((_NCHUNK, _CHUNK), jnp.int32),
            pltpu.VMEM((_NCHUNK, _CHUNK), jnp.int32),
            pltpu.VMEM((_NCHUNK, _CHUNK), jnp.int32),
            pltpu.VMEM((_RPW, _EMB), jnp.float32),
            pltpu.VMEM((_RPW, _EMB), jnp.float32),
            pltpu.VMEM((_RPW, _EMB), jnp.float32),
            pltpu.VMEM((_RPW, _EMB), jnp.float32),
            pltpu.VMEM((_EMB,), jnp.float32),
            pltpu.SemaphoreType.DMA,
        ],
    )
    def k(ii_hbm, ui_hbm, ci_hbm, ti_hbm, ta_hbm, tu_hbm, tc_hbm,
          o_item, o_avg, o_user, o_cat, o_psum,
          v_ii, v_ui, v_ci, v_item, v_avg, v_user, v_cat, v_acc, sem):
        wid = lax.axis_index("s") * _NC + lax.axis_index("c")
        base = wid * _RPW
        r4 = wid * _NCHUNK
        pltpu.sync_copy(ii_hbm.at[pl.ds(r4, _NCHUNK)], v_ii)
        pltpu.sync_copy(ui_hbm.at[pl.ds(r4, _NCHUNK)], v_ui)
        pltpu.sync_copy(ci_hbm.at[pl.ds(r4, _NCHUNK)], v_ci)
        cps = []
        for j in range(_NCHUNK):
            sl = pl.ds(j * _CHUNK, _CHUNK)
            cps.append(pltpu.async_copy(ti_hbm.at[v_ii.at[j]], v_item.at[sl], sem))
            cps.append(pltpu.async_copy(ta_hbm.at[v_ii.at[j]], v_avg.at[sl], sem))
            cps.append(pltpu.async_copy(tu_hbm.at[v_ui.at[j]], v_user.at[sl], sem))
            cps.append(pltpu.async_copy(tc_hbm.at[v_ci.at[j]], v_cat.at[sl], sem))
        for c in cps:
            c.wait()

        wr = [
            pltpu.async_copy(v_item, o_item.at[pl.ds(base, _RPW)], sem),
            pltpu.async_copy(v_user, o_user.at[pl.ds(base, _RPW)], sem),
            pltpu.async_copy(v_cat, o_cat.at[pl.ds(base, _RPW)], sem),
            pltpu.async_copy(v_avg, o_avg.at[pl.ds(base, _RPW)], sem),
        ]

        def body(i, accs):
            r = i * 8
            return tuple(accs[j] + v_avg[r + j] for j in range(8))

        accs = lax.fori_loop(0, _RPW // 8, body,
                             tuple(jnp.zeros((_EMB,), jnp.float32)
                                   for _ in range(8)))
        acc = accs[0]
        for j in range(1, 8):
            acc = acc + accs[j]
        v_acc[...] = acc
        pltpu.sync_copy(v_acc, o_psum.at[wid])
        for c in wr:
            c.wait()

    return k(item_idx, user_idx, cat_idx, t_item, t_avg, t_user, t_cat)


def _tc_dense_body(psum_ref, item_ref, avg_ref, user_ref, cat_ref,
                   sw1_ref, sw2_ref, cw1_ref, cw2_ref,
                   sb1_ref, sb2_ref, cb1_ref, cb2_ref,
                   f0_ref, f1_ref, f2_ref, fb_ref, out_ref):
    total = jnp.sum(psum_ref[...])
    item = item_ref[...]
    user = user_ref[...]
    cat = cat_ref[...]
    user_sel = jnp.where(total == 0.0, user, avg_ref[...])

    def mm(a, b):
        return jax.lax.dot_general(
            a, b, (((1,), (0,)), ((), ())),
            preferred_element_type=jnp.float32)

    h_u = jnp.maximum(mm(user_sel, sw1_ref[...]) + sb1_ref[...], 0.0)
    shift = mm(h_u, sw2_ref[...]) + sb2_ref[...]
    h_c = jnp.maximum(mm(cat, cw1_ref[...]) + cb1_ref[...], 0.0)
    scale = mm(h_c, cw2_ref[...]) + cb2_ref[...]
    warm = scale * item + shift
    tmp = warm * f0_ref[...] + user * f1_ref[...] + cat * f2_ref[...]
    gi = lax.broadcasted_iota(jnp.int32, (128, 8), 0) // 16
    go = lax.broadcasted_iota(jnp.int32, (128, 8), 1)
    sel = jnp.where(gi == go, 1.0, 0.0).astype(jnp.float32)
    logit = mm(tmp, sel) + fb_ref[0, 0]
    out_ref[...] = 1.0 / (1.0 + jnp.exp(-logit))


def kernel(item_id, user_id, item_cat, new_item_emb, avg_users_emb,
           user_emb_table, cat_emb_table, shift_w1, shift_b1, shift_w2,
           shift_b2, scale_w1, scale_b1, scale_w2, scale_b2, final_w,
           final_b):
    g_item, g_avg, g_user, g_cat, psum = _sc_gather(
        item_id.reshape(_B // 128, 128),
        user_id.reshape(_B // 128, 128),
        item_cat.reshape(_B // 128, 128),
        new_item_emb, avg_users_emb, user_emb_table, cat_emb_table)

    rows = _B // 8
    pk = lambda a: a.reshape(rows, 128)
    eye8 = jnp.eye(8, dtype=jnp.float32)
    bd = lambda w: jnp.kron(eye8, w)
    tile8 = lambda v: jnp.tile(v.reshape(-1), 8).reshape(1, 128)

    f = final_w.reshape(3, _EMB)

    blk = 256
    grid = (rows // blk,)
    full = lambda shape: pl.BlockSpec(shape, lambda i: (0, 0))
    out = pl.pallas_call(
        _tc_dense_body,
        grid=grid,
        in_specs=[
            full((_NW * _EMB // 128, 128)),
            pl.BlockSpec((blk, 128), lambda i: (i, 0)),
            pl.BlockSpec((blk, 128), lambda i: (i, 0)),
            pl.BlockSpec((blk, 128), lambda i: (i, 0)),
            pl.BlockSpec((blk, 128), lambda i: (i, 0)),
            full((128, 128)), full((128, 128)),
            full((128, 128)), full((128, 128)),
            full((1, 128)), full((1, 128)), full((1, 128)), full((1, 128)),
            full((1, 128)), full((1, 128)), full((1, 128)),
            full((1, 1)),
        ],
        out_specs=pl.BlockSpec((blk, 8), lambda i: (i, 0)),
        out_shape=jax.ShapeDtypeStruct((rows, 8), jnp.float32),
    )(psum.reshape(_NW * _EMB // 128, 128),
      pk(g_item), pk(g_avg), pk(g_user), pk(g_cat),
      bd(shift_w1), bd(shift_w2), bd(scale_w1), bd(scale_w2),
      tile8(shift_b1), tile8(shift_b2), tile8(scale_b1), tile8(scale_b2),
      tile8(f[0]), tile8(f[1]), tile8(f[2]),
      final_b.reshape(1, 1))
    return out.reshape(_B, 1)
